# Initial kernel scaffold; baseline (speedup 1.0000x reference)
import jax
import jax.numpy as jnp
from jax.experimental import pallas as pl


def kernel(hidden_states, g1, b1, W_down, b_down, g2, b2, memory, W_k, b_k, W_v, b_v, g3, b3, W_up, b_up, layer_id):
    raise NotImplementedError("write your pallas kernel here")


# speedup vs baseline: 1.3697x; 1.3697x over previous
import functools

import jax
import jax.numpy as jnp
from jax.experimental import pallas as pl
from jax.experimental.pallas import tpu as pltpu

_EPS = 1e-12


def _ffn_body(x_ref, g1_ref, b1_ref, wd_ref, bd_ref, g2_ref, b2_ref,
              mem_ref, wk_ref, bk_ref, wv_ref, bv_ref, g3_ref, b3_ref,
              wu_ref, bu_ref, o_ref):
    x = x_ref[...]
    m = jnp.mean(x, axis=-1, keepdims=True)
    xc = x - m
    v = jnp.mean(xc * xc, axis=-1, keepdims=True)
    h = xc * jax.lax.rsqrt(v + _EPS) * g1_ref[...] + b1_ref[...]

    d = jnp.dot(h, wd_ref[...], preferred_element_type=jnp.float32) + bd_ref[...]

    m2 = jnp.mean(d, axis=-1, keepdims=True)
    dc = d - m2
    v2 = jnp.mean(dc * dc, axis=-1, keepdims=True)
    q = dc * jax.lax.rsqrt(v2 + _EPS) * g2_ref[...] + b2_ref[...]

    mem = mem_ref[...]
    key = jnp.dot(mem, wk_ref[...], preferred_element_type=jnp.float32) + bk_ref[...]
    val = jnp.dot(mem, wv_ref[...], preferred_element_type=jnp.float32) + bv_ref[...]

    logits = jax.lax.dot_general(q, key, (((1,), (1,)), ((), ())),
                                 preferred_element_type=jnp.float32)
    logits = logits - jnp.max(logits, axis=-1, keepdims=True)
    e = jnp.exp(logits)
    p = e / jnp.sum(e, axis=-1, keepdims=True)
    mo = jnp.dot(p, val, preferred_element_type=jnp.float32)

    m3 = jnp.mean(mo, axis=-1, keepdims=True)
    mc = mo - m3
    v3 = jnp.mean(mc * mc, axis=-1, keepdims=True)
    y = mc * jax.lax.rsqrt(v3 + _EPS) * g3_ref[...] + b3_ref[...]

    o_ref[...] = jnp.dot(y, wu_ref[...], preferred_element_type=jnp.float32) + bu_ref[...]


@functools.partial(jax.jit, static_argnames=("block_rows", "interpret"))
def _run(x2d, g1, b1, W_down, b_down, g2, b2, memory, W_k, b_k, W_v, b_v,
         g3, b3, W_up, b_up, block_rows=1024, interpret=False):
    n, H = x2d.shape
    D = W_down.shape[1]
    grid = (n // block_rows,)

    def rowspec():
        return pl.BlockSpec((block_rows, H), lambda i: (i, 0))

    def full(a):
        return pl.BlockSpec(a.shape, lambda i: (0,) * a.ndim)

    ins = (g1.reshape(1, -1), b1.reshape(1, -1), W_down, b_down.reshape(1, -1),
           g2.reshape(1, -1), b2.reshape(1, -1), memory, W_k, b_k.reshape(1, -1),
           W_v, b_v.reshape(1, -1), g3.reshape(1, -1), b3.reshape(1, -1),
           W_up, b_up.reshape(1, -1))

    return pl.pallas_call(
        _ffn_body,
        out_shape=jax.ShapeDtypeStruct((n, H), jnp.float32),
        grid=grid,
        in_specs=[rowspec()] + [full(a) for a in ins],
        out_specs=pl.BlockSpec((block_rows, H), lambda i: (i, 0)),
        compiler_params=pltpu.CompilerParams(
            dimension_semantics=("parallel",),
        ),
        name="bert_ffn_memory",
        interpret=interpret,
    )(x2d, *ins)


def kernel(hidden_states, g1, b1, W_down, b_down, g2, b2, memory, W_k, b_k,
           W_v, b_v, g3, b3, W_up, b_up, layer_id):
    B, S, H = hidden_states.shape
    x2d = hidden_states.reshape(B * S, H)
    out = _run(x2d, g1, b1, W_down, b_down, g2, b2, memory, W_k, b_k,
               W_v, b_v, g3, b3, W_up, b_up)
    return out.reshape(B, S, H)
